# baseline (speedup 1.0000x reference)
import functools

import jax
import jax.numpy as jnp
from jax import lax
from jax.experimental import pallas as pl
from jax.experimental.pallas import tpu as pltpu


def _bilstm_kernel(
    x_ref,
    len_ref,
    wif_ref,
    wib_ref,
    bf_ref,
    bb_ref,
    whf_ref,
    whb_ref,
    w1f_ref,
    w1b_ref,
    b1_ref,
    bns_ref,
    bnt_ref,
    lng_ref,
    lnb_ref,
    w2_ref,
    b2_ref,
    out_ref,
    *,
    h_real,
):
    T, BT, _ = x_ref.shape
    H = whf_ref.shape[0]

    len_bh = len_ref[...]
    bfv = bf_ref[...]
    bbv = bb_ref[...]
    whf = whf_ref[...]
    whb = whb_ref[...]
    wif = wif_ref[...]
    wib = wib_ref[...]

    zeros = jnp.zeros((BT, H), jnp.float32)
    hf, cf, af = zeros, zeros, zeros
    hb, cb, ab = zeros, zeros, zeros

    def sig(v):
        return 0.5 + 0.5 * jnp.tanh(0.5 * v)

    def cell(gates, c):
        i = sig(gates[:, 0:H])
        f = sig(gates[:, H:2 * H])
        g = jnp.tanh(gates[:, 2 * H:3 * H])
        o = sig(gates[:, 3 * H:4 * H])
        c_new = f * c + i * g
        h_new = o * jnp.tanh(c_new)
        return h_new, c_new

    for s in range(T):
        tb = T - 1 - s
        gf = (jnp.dot(x_ref[s], wif, preferred_element_type=jnp.float32)
              + jnp.dot(hf.astype(jnp.bfloat16), whf,
                        preferred_element_type=jnp.float32) + bfv)
        gb = (jnp.dot(x_ref[tb], wib, preferred_element_type=jnp.float32)
              + jnp.dot(hb.astype(jnp.bfloat16), whb,
                        preferred_element_type=jnp.float32) + bbv)
        hf, cf = cell(gf, cf)
        hb, cb = cell(gb, cb)
        af = af + jnp.where(len_bh > s, hf, 0.0)
        ab = ab + jnp.where(len_bh > tb, hb, 0.0)

    h1 = (jnp.dot(af.astype(jnp.bfloat16), w1f_ref[...],
                  preferred_element_type=jnp.float32)
          + jnp.dot(ab.astype(jnp.bfloat16), w1b_ref[...],
                    preferred_element_type=jnp.float32)
          + b1_ref[...])
    h1 = jnp.maximum(h1, 0.0)

    bn = h1 * bns_ref[...] + bnt_ref[...]

    if h_real == H:
        inv_h = jnp.float32(1.0 / h_real)
        mu = jnp.sum(bn, axis=-1, keepdims=True) * inv_h
        cen = bn - mu
    else:
        fmask = (lax.broadcasted_iota(jnp.int32, (1, H), 1)
                 < h_real).astype(jnp.float32)
        inv_h = jnp.float32(1.0 / h_real)
        mu = jnp.sum(bn * fmask, axis=-1, keepdims=True) * inv_h
        cen = (bn - mu) * fmask
    var = jnp.sum(cen * cen, axis=-1, keepdims=True) * inv_h
    ln = cen * lax.rsqrt(var + jnp.float32(1e-5)) * lng_ref[...] + lnb_ref[...]

    out_ref[...] = (jnp.dot(ln, w2_ref[...],
                            preferred_element_type=jnp.float32) + b2_ref[...])


def _round_up(n, m):
    return ((n + m - 1) // m) * m


@jax.jit
def _forward(x, mask, wih_f, whh_f, b_f, wih_b, whh_b, b_b, w1, b1,
             bn_g, bn_b, bn_m, bn_v, ln_g, ln_b, w2, b2):
    x = jnp.asarray(x, jnp.float32)
    mask = jnp.asarray(mask, jnp.float32)
    B, T, D = x.shape
    H = whh_f.shape[0]

    B_tile = next((t for t in (512, 256, 128) if B % t == 0), 128)
    B_p = _round_up(B, B_tile)
    n_b = B_p // B_tile

    x_tbd = jnp.transpose(x, (1, 0, 2)).astype(jnp.bfloat16)
    x_tbd = jnp.pad(x_tbd, ((0, 0), (0, B_p - B), (0, 0)))

    lengths = jnp.sum(mask, axis=1).astype(jnp.int32)
    lengths = jnp.pad(lengths, (0, B_p - B))
    len_bh = jnp.broadcast_to(lengths[:, None], (B_p, H)).astype(jnp.int32)

    bf16 = jnp.bfloat16
    wif = wih_f.astype(bf16)
    wib = wih_b.astype(bf16)
    whf = whh_f.astype(bf16)
    whb = whh_b.astype(bf16)
    w1f = w1[:H].astype(bf16)
    w1b = w1[H:].astype(bf16)

    eps = 1e-5
    bn_scale = bn_g * lax.rsqrt(bn_v + eps)
    bn_shift = bn_b - bn_m * bn_scale

    body = functools.partial(_bilstm_kernel, h_real=H)

    def full(shape):
        return pl.BlockSpec(shape, lambda b, _n=len(shape): (0,) * _n)

    out = pl.pallas_call(
        body,
        out_shape=jax.ShapeDtypeStruct((B_p, 1), jnp.float32),
        grid=(n_b,),
        in_specs=[
            pl.BlockSpec((T, B_tile, D), lambda b: (0, b, 0)),
            pl.BlockSpec((B_tile, H), lambda b: (b, 0)),
            full((D, 4 * H)),
            full((D, 4 * H)),
            full((1, 4 * H)),
            full((1, 4 * H)),
            full((H, 4 * H)),
            full((H, 4 * H)),
            full((H, H)),
            full((H, H)),
            full((1, H)),
            full((1, H)),
            full((1, H)),
            full((1, H)),
            full((1, H)),
            full((H, 1)),
            full((1, 1)),
        ],
        out_specs=pl.BlockSpec((B_tile, 1), lambda b: (b, 0)),
        compiler_params=pltpu.CompilerParams(
            dimension_semantics=("parallel",),
        ),
    )(x_tbd, len_bh, wif, wib, b_f, b_b, whf, whb, w1f, w1b, b1,
      bn_scale, bn_shift, ln_g, ln_b, w2, b2)

    return out[:B]


def kernel(x, mask, wih_f, whh_f, b_f, wih_b, whh_b, b_b, w1, b1,
           bn_g, bn_b, bn_m, bn_v, ln_g, ln_b, w2, b2):
    return _forward(x, mask, wih_f, whh_f, b_f, wih_b, whh_b, b_b, w1, b1,
                    bn_g, bn_b, bn_m, bn_v, ln_g, ln_b, w2, b2)


# speedup vs baseline: 1.2424x; 1.2424x over previous
import functools

import jax
import jax.numpy as jnp
from jax import lax
from jax.experimental import pallas as pl
from jax.experimental.pallas import tpu as pltpu


def _bilstm_kernel(
    x_ref,
    len_ref,
    wif_ref,
    wib_ref,
    bf_ref,
    bb_ref,
    whf_ref,
    whb_ref,
    w1f_ref,
    w1b_ref,
    b1_ref,
    bns_ref,
    bnt_ref,
    lng_ref,
    lnb_ref,
    w2_ref,
    b2_ref,
    out_ref,
    *,
    h_real,
):
    T, BT, _ = x_ref.shape
    H = whf_ref.shape[0]

    len_bh = len_ref[...]
    bfv = bf_ref[...]
    bbv = bb_ref[...]
    whf = whf_ref[...]
    whb = whb_ref[...]
    wif = wif_ref[...]
    wib = wib_ref[...]

    zeros = jnp.zeros((BT, H), jnp.float32)
    hf, cf, af = zeros, zeros, zeros
    hb, cb, ab = zeros, zeros, zeros

    def sig(v):
        return 0.5 + 0.5 * jnp.tanh(0.5 * v)

    def cell(gates, c):
        i = sig(gates[:, 0:H])
        f = sig(gates[:, H:2 * H])
        g = jnp.tanh(gates[:, 2 * H:3 * H])
        o = sig(gates[:, 3 * H:4 * H])
        c_new = f * c + i * g
        h_new = o * jnp.tanh(c_new)
        return h_new, c_new

    for s in range(T):
        tb = T - 1 - s
        gf = (jnp.dot(x_ref[s], wif, preferred_element_type=jnp.float32)
              + jnp.dot(hf.astype(jnp.bfloat16), whf,
                        preferred_element_type=jnp.float32) + bfv)
        gb = (jnp.dot(x_ref[tb], wib, preferred_element_type=jnp.float32)
              + jnp.dot(hb.astype(jnp.bfloat16), whb,
                        preferred_element_type=jnp.float32) + bbv)
        hf, cf = cell(gf, cf)
        hb, cb = cell(gb, cb)
        af = af + jnp.where(len_bh > s, hf, 0.0)
        ab = ab + jnp.where(len_bh > tb, hb, 0.0)

    h1 = (jnp.dot(af.astype(jnp.bfloat16), w1f_ref[...],
                  preferred_element_type=jnp.float32)
          + jnp.dot(ab.astype(jnp.bfloat16), w1b_ref[...],
                    preferred_element_type=jnp.float32)
          + b1_ref[...])
    h1 = jnp.maximum(h1, 0.0)

    bn = h1 * bns_ref[...] + bnt_ref[...]

    if h_real == H:
        inv_h = jnp.float32(1.0 / h_real)
        mu = jnp.sum(bn, axis=-1, keepdims=True) * inv_h
        cen = bn - mu
    else:
        fmask = (lax.broadcasted_iota(jnp.int32, (1, H), 1)
                 < h_real).astype(jnp.float32)
        inv_h = jnp.float32(1.0 / h_real)
        mu = jnp.sum(bn * fmask, axis=-1, keepdims=True) * inv_h
        cen = (bn - mu) * fmask
    var = jnp.sum(cen * cen, axis=-1, keepdims=True) * inv_h
    ln = cen * lax.rsqrt(var + jnp.float32(1e-5)) * lng_ref[...] + lnb_ref[...]

    out_ref[...] = (jnp.dot(ln, w2_ref[...],
                            preferred_element_type=jnp.float32) + b2_ref[...])


def _round_up(n, m):
    return ((n + m - 1) // m) * m


@jax.jit
def _forward(x, mask, wih_f, whh_f, b_f, wih_b, whh_b, b_b, w1, b1,
             bn_g, bn_b, bn_m, bn_v, ln_g, ln_b, w2, b2):
    x = jnp.asarray(x, jnp.float32)
    mask = jnp.asarray(mask, jnp.float32)
    B, T, D = x.shape
    H = whh_f.shape[0]

    B_tile = 128
    B_p = _round_up(B, B_tile)
    n_b = B_p // B_tile

    x_tbd = jnp.transpose(x, (1, 0, 2)).astype(jnp.bfloat16)
    x_tbd = jnp.pad(x_tbd, ((0, 0), (0, B_p - B), (0, 0)))

    lengths = jnp.sum(mask, axis=1).astype(jnp.int32)
    lengths = jnp.pad(lengths, (0, B_p - B))
    len_bh = jnp.broadcast_to(lengths[:, None], (B_p, H)).astype(jnp.int32)

    bf16 = jnp.bfloat16
    wif = wih_f.astype(bf16)
    wib = wih_b.astype(bf16)
    whf = whh_f.astype(bf16)
    whb = whh_b.astype(bf16)
    w1f = w1[:H].astype(bf16)
    w1b = w1[H:].astype(bf16)

    eps = 1e-5
    bn_scale = bn_g * lax.rsqrt(bn_v + eps)
    bn_shift = bn_b - bn_m * bn_scale

    body = functools.partial(_bilstm_kernel, h_real=H)

    def full(shape):
        return pl.BlockSpec(shape, lambda b, _n=len(shape): (0,) * _n)

    out = pl.pallas_call(
        body,
        out_shape=jax.ShapeDtypeStruct((B_p, 1), jnp.float32),
        grid=(n_b,),
        in_specs=[
            pl.BlockSpec((T, B_tile, D), lambda b: (0, b, 0)),
            pl.BlockSpec((B_tile, H), lambda b: (b, 0)),
            full((D, 4 * H)),
            full((D, 4 * H)),
            full((1, 4 * H)),
            full((1, 4 * H)),
            full((H, 4 * H)),
            full((H, 4 * H)),
            full((H, H)),
            full((H, H)),
            full((1, H)),
            full((1, H)),
            full((1, H)),
            full((1, H)),
            full((1, H)),
            full((H, 1)),
            full((1, 1)),
        ],
        out_specs=pl.BlockSpec((B_tile, 1), lambda b: (b, 0)),
        compiler_params=pltpu.CompilerParams(
            dimension_semantics=("parallel",),
        ),
    )(x_tbd, len_bh, wif, wib, b_f, b_b, whf, whb, w1f, w1b, b1,
      bn_scale, bn_shift, ln_g, ln_b, w2, b2)

    return out[:B]


def kernel(x, mask, wih_f, whh_f, b_f, wih_b, whh_b, b_b, w1, b1,
           bn_g, bn_b, bn_m, bn_v, ln_g, ln_b, w2, b2):
    return _forward(x, mask, wih_f, whh_f, b_f, wih_b, whh_b, b_b, w1, b1,
                    bn_g, bn_b, bn_m, bn_v, ln_g, ln_b, w2, b2)


# speedup vs baseline: 1.2713x; 1.0232x over previous
import functools

import jax
import jax.numpy as jnp
from jax import lax
from jax.experimental import pallas as pl
from jax.experimental.pallas import tpu as pltpu


def _bilstm_kernel(
    x_ref,
    len_ref,
    wf_ref,
    wb_ref,
    w1f_ref,
    w1b_ref,
    b1_ref,
    bns_ref,
    bnt_ref,
    lng_ref,
    lnb_ref,
    w2_ref,
    b2_ref,
    out_ref,
    lhsf_ref,
    lhsb_ref,
    *,
    h_real,
):
    T, BT, D = x_ref.shape
    H = w1f_ref.shape[0]
    K = wf_ref.shape[0]

    len_bh = len_ref[...]
    wf = wf_ref[...]
    wb = wb_ref[...]

    tail = (lax.broadcasted_iota(jnp.int32, (BT, K - D - H), 1) == 0)
    tail = tail.astype(jnp.bfloat16)
    lhsf_ref[:, D + H:] = tail
    lhsb_ref[:, D + H:] = tail

    zeros = jnp.zeros((BT, H), jnp.float32)
    hf, cf, af = zeros, zeros, zeros
    hb, cb, ab = zeros, zeros, zeros
    lhsf_ref[:, D:D + H] = jnp.zeros((BT, H), jnp.bfloat16)
    lhsb_ref[:, D:D + H] = jnp.zeros((BT, H), jnp.bfloat16)

    def cell(gates, c):
        ti = jnp.tanh(gates[:, 0:H])
        tf = jnp.tanh(gates[:, H:2 * H])
        g = jnp.tanh(gates[:, 2 * H:3 * H])
        to = jnp.tanh(gates[:, 3 * H:4 * H])
        c_new = 0.5 * ((c + g) + (tf * c + ti * g))
        tc = jnp.tanh(c_new)
        h_new = 0.5 * (tc + to * tc)
        return h_new, c_new

    for s in range(T):
        tb = T - 1 - s
        lhsf_ref[:, 0:D] = x_ref[s]
        lhsb_ref[:, 0:D] = x_ref[tb]
        gf = jnp.dot(lhsf_ref[...], wf, preferred_element_type=jnp.float32)
        gb = jnp.dot(lhsb_ref[...], wb, preferred_element_type=jnp.float32)
        hf, cf = cell(gf, cf)
        hb, cb = cell(gb, cb)
        if s + 1 < T:
            lhsf_ref[:, D:D + H] = hf.astype(jnp.bfloat16)
            lhsb_ref[:, D:D + H] = hb.astype(jnp.bfloat16)
        af = af + jnp.where(len_bh > s, hf, 0.0)
        ab = ab + jnp.where(len_bh > tb, hb, 0.0)

    h1 = (jnp.dot(af.astype(jnp.bfloat16), w1f_ref[...],
                  preferred_element_type=jnp.float32)
          + jnp.dot(ab.astype(jnp.bfloat16), w1b_ref[...],
                    preferred_element_type=jnp.float32)
          + b1_ref[...])
    h1 = jnp.maximum(h1, 0.0)

    bn = h1 * bns_ref[...] + bnt_ref[...]

    if h_real == H:
        inv_h = jnp.float32(1.0 / h_real)
        mu = jnp.sum(bn, axis=-1, keepdims=True) * inv_h
        cen = bn - mu
    else:
        fmask = (lax.broadcasted_iota(jnp.int32, (1, H), 1)
                 < h_real).astype(jnp.float32)
        inv_h = jnp.float32(1.0 / h_real)
        mu = jnp.sum(bn * fmask, axis=-1, keepdims=True) * inv_h
        cen = (bn - mu) * fmask
    var = jnp.sum(cen * cen, axis=-1, keepdims=True) * inv_h
    ln = cen * lax.rsqrt(var + jnp.float32(1e-5)) * lng_ref[...] + lnb_ref[...]

    out_ref[...] = (jnp.dot(ln, w2_ref[...],
                            preferred_element_type=jnp.float32) + b2_ref[...])


def _round_up(n, m):
    return ((n + m - 1) // m) * m


@jax.jit
def _forward(x, mask, wih_f, whh_f, b_f, wih_b, whh_b, b_b, w1, b1,
             bn_g, bn_b, bn_m, bn_v, ln_g, ln_b, w2, b2):
    x = jnp.asarray(x, jnp.float32)
    mask = jnp.asarray(mask, jnp.float32)
    B, T, D = x.shape
    H = whh_f.shape[0]
    K = _round_up(D + H + 1, 128)

    B_tile = 256 if B % 256 == 0 else 128
    B_p = _round_up(B, B_tile)
    n_b = B_p // B_tile

    x_tbd = jnp.transpose(x, (1, 0, 2)).astype(jnp.bfloat16)
    x_tbd = jnp.pad(x_tbd, ((0, 0), (0, B_p - B), (0, 0)))

    lengths = jnp.sum(mask, axis=1).astype(jnp.int32)
    lengths = jnp.pad(lengths, (0, B_p - B))
    len_bh = jnp.broadcast_to(lengths[:, None], (B_p, H)).astype(jnp.int32)

    bf16 = jnp.bfloat16

    gate_scale = jnp.concatenate(
        [jnp.full((1, H), 0.5, jnp.float32),
         jnp.full((1, H), 0.5, jnp.float32),
         jnp.ones((1, H), jnp.float32),
         jnp.full((1, H), 0.5, jnp.float32)], axis=1)

    def pack(wih, whh, b):
        w = jnp.concatenate(
            [wih, whh, b, jnp.zeros((K - D - H - 1, 4 * H), jnp.float32)],
            axis=0)
        return (w * gate_scale).astype(bf16)

    wf = pack(wih_f, whh_f, b_f)
    wb = pack(wih_b, whh_b, b_b)
    w1f = w1[:H].astype(bf16)
    w1b = w1[H:].astype(bf16)

    eps = 1e-5
    bn_scale = bn_g * lax.rsqrt(bn_v + eps)
    bn_shift = bn_b - bn_m * bn_scale

    body = functools.partial(_bilstm_kernel, h_real=H)

    def full(shape):
        return pl.BlockSpec(shape, lambda b, _n=len(shape): (0,) * _n)

    out = pl.pallas_call(
        body,
        out_shape=jax.ShapeDtypeStruct((B_p, 1), jnp.float32),
        grid=(n_b,),
        in_specs=[
            pl.BlockSpec((T, B_tile, D), lambda b: (0, b, 0)),
            pl.BlockSpec((B_tile, H), lambda b: (b, 0)),
            full((K, 4 * H)),
            full((K, 4 * H)),
            full((H, H)),
            full((H, H)),
            full((1, H)),
            full((1, H)),
            full((1, H)),
            full((1, H)),
            full((1, H)),
            full((H, 1)),
            full((1, 1)),
        ],
        out_specs=pl.BlockSpec((B_tile, 1), lambda b: (b, 0)),
        scratch_shapes=[
            pltpu.VMEM((B_tile, K), bf16),
            pltpu.VMEM((B_tile, K), bf16),
        ],
        compiler_params=pltpu.CompilerParams(
            dimension_semantics=("parallel",),
        ),
    )(x_tbd, len_bh, wf, wb, w1f, w1b, b1,
      bn_scale, bn_shift, ln_g, ln_b, w2, b2)

    return out[:B]


def kernel(x, mask, wih_f, whh_f, b_f, wih_b, whh_b, b_b, w1, b1,
           bn_g, bn_b, bn_m, bn_v, ln_g, ln_b, w2, b2):
    return _forward(x, mask, wih_f, whh_f, b_f, wih_b, whh_b, b_b, w1, b1,
                    bn_g, bn_b, bn_m, bn_v, ln_g, ln_b, w2, b2)


# speedup vs baseline: 1.3136x; 1.0333x over previous
import functools

import jax
import jax.numpy as jnp
from jax import lax
from jax.experimental import pallas as pl
from jax.experimental.pallas import tpu as pltpu


def _bilstm_kernel(
    x_ref,
    len_ref,
    wif_ref,
    wib_ref,
    bf_ref,
    bb_ref,
    whf_ref,
    whb_ref,
    w1f_ref,
    w1b_ref,
    b1_ref,
    bns_ref,
    bnt_ref,
    lng_ref,
    lnb_ref,
    w2_ref,
    b2_ref,
    out_ref,
    *,
    h_real,
):
    T, BT, _ = x_ref.shape
    H = whf_ref.shape[0]

    len_bh = len_ref[...]
    bfv = bf_ref[...]
    bbv = bb_ref[...]
    whf = whf_ref[...]
    whb = whb_ref[...]
    wif = wif_ref[...]
    wib = wib_ref[...]

    zeros = jnp.zeros((BT, H), jnp.float32)
    hf, cf, af = zeros, zeros, zeros
    hb, cb, ab = zeros, zeros, zeros

    def cell(gates, c):
        ti = jnp.tanh(gates[:, 0:H])
        tf = jnp.tanh(gates[:, H:2 * H])
        g = jnp.tanh(gates[:, 2 * H:3 * H])
        to = jnp.tanh(gates[:, 3 * H:4 * H])
        c_new = 0.5 * ((c + g) + (tf * c + ti * g))
        tc = jnp.tanh(c_new)
        h_new = 0.5 * (tc + to * tc)
        return h_new, c_new

    for s in range(T):
        tb = T - 1 - s
        gf = (jnp.dot(x_ref[s], wif, preferred_element_type=jnp.float32)
              + jnp.dot(hf.astype(jnp.bfloat16), whf,
                        preferred_element_type=jnp.float32) + bfv)
        gb = (jnp.dot(x_ref[tb], wib, preferred_element_type=jnp.float32)
              + jnp.dot(hb.astype(jnp.bfloat16), whb,
                        preferred_element_type=jnp.float32) + bbv)
        hf, cf = cell(gf, cf)
        hb, cb = cell(gb, cb)
        af = af + jnp.where(len_bh > s, hf, 0.0)
        ab = ab + jnp.where(len_bh > tb, hb, 0.0)

    h1 = (jnp.dot(af.astype(jnp.bfloat16), w1f_ref[...],
                  preferred_element_type=jnp.float32)
          + jnp.dot(ab.astype(jnp.bfloat16), w1b_ref[...],
                    preferred_element_type=jnp.float32)
          + b1_ref[...])
    h1 = jnp.maximum(h1, 0.0)

    bn = h1 * bns_ref[...] + bnt_ref[...]

    if h_real == H:
        inv_h = jnp.float32(1.0 / h_real)
        mu = jnp.sum(bn, axis=-1, keepdims=True) * inv_h
        cen = bn - mu
    else:
        fmask = (lax.broadcasted_iota(jnp.int32, (1, H), 1)
                 < h_real).astype(jnp.float32)
        inv_h = jnp.float32(1.0 / h_real)
        mu = jnp.sum(bn * fmask, axis=-1, keepdims=True) * inv_h
        cen = (bn - mu) * fmask
    var = jnp.sum(cen * cen, axis=-1, keepdims=True) * inv_h
    ln = cen * lax.rsqrt(var + jnp.float32(1e-5)) * lng_ref[...] + lnb_ref[...]

    out_ref[...] = (jnp.dot(ln, w2_ref[...],
                            preferred_element_type=jnp.float32) + b2_ref[...])


def _round_up(n, m):
    return ((n + m - 1) // m) * m


@jax.jit
def _forward(x, mask, wih_f, whh_f, b_f, wih_b, whh_b, b_b, w1, b1,
             bn_g, bn_b, bn_m, bn_v, ln_g, ln_b, w2, b2):
    x = jnp.asarray(x, jnp.float32)
    mask = jnp.asarray(mask, jnp.float32)
    B, T, D = x.shape
    H = whh_f.shape[0]

    B_tile = 256 if B % 256 == 0 else 128
    B_p = _round_up(B, B_tile)
    n_b = B_p // B_tile

    x_tbd = jnp.transpose(x, (1, 0, 2)).astype(jnp.bfloat16)
    x_tbd = jnp.pad(x_tbd, ((0, 0), (0, B_p - B), (0, 0)))

    lengths = jnp.sum(mask, axis=1).astype(jnp.int32)
    lengths = jnp.pad(lengths, (0, B_p - B))
    len_bh = jnp.broadcast_to(lengths[:, None], (B_p, H)).astype(jnp.int32)

    bf16 = jnp.bfloat16

    gate_scale = jnp.concatenate(
        [jnp.full((1, H), 0.5, jnp.float32),
         jnp.full((1, H), 0.5, jnp.float32),
         jnp.ones((1, H), jnp.float32),
         jnp.full((1, H), 0.5, jnp.float32)], axis=1)

    wif = (wih_f * gate_scale).astype(bf16)
    wib = (wih_b * gate_scale).astype(bf16)
    whf = (whh_f * gate_scale).astype(bf16)
    whb = (whh_b * gate_scale).astype(bf16)
    bfs = b_f * gate_scale
    bbs = b_b * gate_scale
    w1f = w1[:H].astype(bf16)
    w1b = w1[H:].astype(bf16)

    eps = 1e-5
    bn_scale = bn_g * lax.rsqrt(bn_v + eps)
    bn_shift = bn_b - bn_m * bn_scale

    body = functools.partial(_bilstm_kernel, h_real=H)

    def full(shape):
        return pl.BlockSpec(shape, lambda b, _n=len(shape): (0,) * _n)

    out = pl.pallas_call(
        body,
        out_shape=jax.ShapeDtypeStruct((B_p, 1), jnp.float32),
        grid=(n_b,),
        in_specs=[
            pl.BlockSpec((T, B_tile, D), lambda b: (0, b, 0)),
            pl.BlockSpec((B_tile, H), lambda b: (b, 0)),
            full((D, 4 * H)),
            full((D, 4 * H)),
            full((1, 4 * H)),
            full((1, 4 * H)),
            full((H, 4 * H)),
            full((H, 4 * H)),
            full((H, H)),
            full((H, H)),
            full((1, H)),
            full((1, H)),
            full((1, H)),
            full((1, H)),
            full((1, H)),
            full((H, 1)),
            full((1, 1)),
        ],
        out_specs=pl.BlockSpec((B_tile, 1), lambda b: (b, 0)),
        compiler_params=pltpu.CompilerParams(
            dimension_semantics=("parallel",),
        ),
    )(x_tbd, len_bh, wif, wib, bfs, bbs, whf, whb, w1f, w1b, b1,
      bn_scale, bn_shift, ln_g, ln_b, w2, b2)

    return out[:B]


def kernel(x, mask, wih_f, whh_f, b_f, wih_b, whh_b, b_b, w1, b1,
           bn_g, bn_b, bn_m, bn_v, ln_g, ln_b, w2, b2):
    return _forward(x, mask, wih_f, whh_f, b_f, wih_b, whh_b, b_b, w1, b1,
                    bn_g, bn_b, bn_m, bn_v, ln_g, ln_b, w2, b2)
